# Initial kernel scaffold; baseline (speedup 1.0000x reference)
import functools

import jax
import jax.numpy as jnp
from jax import lax
from jax.experimental import pallas as pl
from jax.experimental.pallas import tpu as pltpu
from jax.experimental.pallas import tpu_sc as plsc

VOCAB = 100000
NUM_SENSE = 3
EMB_DIM = 64
ROW = NUM_SENSE * EMB_DIM

NUM_CORES = 2
NUM_SUBCORES = 16
NW = NUM_CORES * NUM_SUBCORES


def _sc_gather(table, idx):
    B = idx.shape[0]
    b_per_w = B // NW
    mesh = plsc.VectorSubcoreMesh(core_axis_name="c", subcore_axis_name="s")

    @functools.partial(
        pl.kernel,
        mesh=mesh,
        out_type=jax.ShapeDtypeStruct((B, ROW), jnp.float32),
        scratch_types=[
            pltpu.VMEM((b_per_w,), jnp.int32),
            pltpu.VMEM((b_per_w, ROW), jnp.float32),
            pltpu.SemaphoreType.DMA,
        ],
    )
    def k(table_hbm, idx_hbm, out_hbm, idx_v, rows_v, sem):
        wid = lax.axis_index("s") * NUM_CORES + lax.axis_index("c")
        base = wid * b_per_w
        pltpu.sync_copy(idx_hbm.at[pl.ds(base, b_per_w)], idx_v)
        pltpu.async_copy(table_hbm.at[idx_v], rows_v, sem).wait()
        pltpu.sync_copy(rows_v, out_hbm.at[pl.ds(base, b_per_w)])

    return k(table, idx)


def _tc_combine(emb_rows, dis_rows, ctx):
    B = ctx.shape[0]
    BLK = 1024

    def body(emb_ref, dis_ref, ctx_ref, out_ref):
        e = emb_ref[...]
        d = dis_ref[...]
        c = ctx_ref[...]
        a0 = jnp.sum(d[:, 0:EMB_DIM] * c, axis=1, keepdims=True)
        a1 = jnp.sum(d[:, EMB_DIM : 2 * EMB_DIM] * c, axis=1, keepdims=True)
        a2 = jnp.sum(d[:, 2 * EMB_DIM : 3 * EMB_DIM] * c, axis=1, keepdims=True)
        m = jnp.maximum(a0, jnp.maximum(a1, a2))
        e0 = jnp.exp(a0 - m)
        e1 = jnp.exp(a1 - m)
        e2 = jnp.exp(a2 - m)
        den = e0 + e1 + e2
        out_ref[...] = (
            e0 * e[:, 0:EMB_DIM]
            + e1 * e[:, EMB_DIM : 2 * EMB_DIM]
            + e2 * e[:, 2 * EMB_DIM : 3 * EMB_DIM]
        ) / den

    return pl.pallas_call(
        body,
        grid=(B // BLK,),
        in_specs=[
            pl.BlockSpec((BLK, ROW), lambda i: (i, 0)),
            pl.BlockSpec((BLK, ROW), lambda i: (i, 0)),
            pl.BlockSpec((BLK, EMB_DIM), lambda i: (i, 0)),
        ],
        out_specs=pl.BlockSpec((BLK, EMB_DIM), lambda i: (i, 0)),
        out_shape=jax.ShapeDtypeStruct((B, EMB_DIM), jnp.float32),
    )(emb_rows, dis_rows, ctx)


def kernel(word_ids, ctx, emb_table, disamb_table):
    idx = word_ids.astype(jnp.int32)
    emb2 = emb_table.reshape(VOCAB, ROW)
    dis2 = disamb_table.reshape(VOCAB, ROW)
    emb_rows = _sc_gather(emb2, idx)
    dis_rows = _sc_gather(dis2, idx)
    return _tc_combine(emb_rows, dis_rows, ctx)


# speedup vs baseline: 3.0489x; 3.0489x over previous
import functools

import jax
import jax.numpy as jnp
from jax import lax
from jax.experimental import pallas as pl
from jax.experimental.pallas import tpu as pltpu
from jax.experimental.pallas import tpu_sc as plsc

VOCAB = 100000
NUM_SENSE = 3
EMB_DIM = 64
ROW = NUM_SENSE * EMB_DIM

NUM_CORES = 2
NUM_SUBCORES = 16
NW = NUM_CORES * NUM_SUBCORES


def _sc_gather(table, idx):
    B = idx.shape[0]
    b_per_w = B // NW
    mesh = plsc.VectorSubcoreMesh(core_axis_name="c", subcore_axis_name="s")

    @functools.partial(
        pl.kernel,
        mesh=mesh,
        compiler_params=pltpu.CompilerParams(use_tc_tiling_on_sc=False),
        out_type=jax.ShapeDtypeStruct((B, ROW), jnp.float32),
        scratch_types=[
            pltpu.VMEM((b_per_w,), jnp.int32),
            pltpu.VMEM((b_per_w, ROW), jnp.float32),
            pltpu.SemaphoreType.DMA,
        ],
    )
    def k(table_hbm, idx_hbm, out_hbm, idx_v, rows_v, sem):
        wid = lax.axis_index("s") * NUM_CORES + lax.axis_index("c")
        base = wid * b_per_w
        pltpu.sync_copy(idx_hbm.at[pl.ds(base, b_per_w)], idx_v)
        pltpu.async_copy(table_hbm.at[idx_v], rows_v, sem).wait()
        pltpu.sync_copy(rows_v, out_hbm.at[pl.ds(base, b_per_w)])

    return k(table, idx)


def _tc_combine(emb_rows, dis_rows, ctx):
    B = ctx.shape[0]
    BLK = 1024

    def body(emb_ref, dis_ref, ctx_ref, out_ref):
        e = emb_ref[...]
        d = dis_ref[...]
        c = ctx_ref[...]
        a0 = jnp.sum(d[:, 0:EMB_DIM] * c, axis=1, keepdims=True)
        a1 = jnp.sum(d[:, EMB_DIM : 2 * EMB_DIM] * c, axis=1, keepdims=True)
        a2 = jnp.sum(d[:, 2 * EMB_DIM : 3 * EMB_DIM] * c, axis=1, keepdims=True)
        m = jnp.maximum(a0, jnp.maximum(a1, a2))
        e0 = jnp.exp(a0 - m)
        e1 = jnp.exp(a1 - m)
        e2 = jnp.exp(a2 - m)
        den = e0 + e1 + e2
        out_ref[...] = (
            e0 * e[:, 0:EMB_DIM]
            + e1 * e[:, EMB_DIM : 2 * EMB_DIM]
            + e2 * e[:, 2 * EMB_DIM : 3 * EMB_DIM]
        ) / den

    return pl.pallas_call(
        body,
        grid=(B // BLK,),
        in_specs=[
            pl.BlockSpec((BLK, ROW), lambda i: (i, 0)),
            pl.BlockSpec((BLK, ROW), lambda i: (i, 0)),
            pl.BlockSpec((BLK, EMB_DIM), lambda i: (i, 0)),
        ],
        out_specs=pl.BlockSpec((BLK, EMB_DIM), lambda i: (i, 0)),
        out_shape=jax.ShapeDtypeStruct((B, EMB_DIM), jnp.float32),
    )(emb_rows, dis_rows, ctx)


def kernel(word_ids, ctx, emb_table, disamb_table):
    idx = word_ids.astype(jnp.int32)
    emb2 = emb_table.reshape(VOCAB, ROW)
    dis2 = disamb_table.reshape(VOCAB, ROW)
    emb_rows = _sc_gather(emb2, idx)
    dis_rows = _sc_gather(dis2, idx)
    return _tc_combine(emb_rows, dis_rows, ctx)
